# baseline (speedup 1.0000x reference)
import functools

import jax
import jax.numpy as jnp
from jax import lax
from jax.experimental import pallas as pl
from jax.experimental.pallas import tpu as pltpu
from jax.experimental.pallas import tpu_sc as plsc

_NUM_WORKERS = 32
_LANES = 16
_CHUNK = 32
_NBUF = 2


def _paged_update(tok_k, tok_v, slot_mapping, k_cache, v_cache):
    n_tok = tok_k.shape[0]
    num_slots = k_cache.shape[0]
    n_heads, head_dim = k_cache.shape[1], k_cache.shape[2]
    span = num_slots // _NUM_WORKERS
    cchunks = span // _CHUNK
    total = 2 * cchunks
    mesh = plsc.VectorSubcoreMesh(core_axis_name="c", subcore_axis_name="s")

    @functools.partial(
        pl.kernel,
        mesh=mesh,
        out_type=(
            jax.ShapeDtypeStruct(k_cache.shape, k_cache.dtype),
            jax.ShapeDtypeStruct(v_cache.shape, v_cache.dtype),
        ),
        scratch_types=[
            pltpu.VMEM((_NBUF, _CHUNK, n_heads, head_dim), jnp.float32),
            pltpu.VMEM((n_tok,), jnp.int32),
            pltpu.SemaphoreType.DMA((_NBUF,)),
            pltpu.SemaphoreType.DMA((_NBUF,)),
        ],
    )
    def body(tok_k_hbm, tok_v_hbm, slot_hbm, kc_in, vc_in, kc_out, vc_out,
             buf, idx_v, sem_in, sem_out):
        wid = lax.axis_index("s") * 2 + lax.axis_index("c")
        base = wid * span
        srcs = (kc_in, vc_in)
        dsts = (kc_out, vc_out)

        def in_copy(c, b):
            rows = pl.ds(base + (c // 2) * _CHUNK, _CHUNK)
            return pltpu.make_async_copy(
                srcs[c % 2].at[rows], buf.at[b], sem_in.at[b])

        def out_copy(c, b):
            rows = pl.ds(base + (c // 2) * _CHUNK, _CHUNK)
            return pltpu.make_async_copy(
                buf.at[b], dsts[c % 2].at[rows], sem_out.at[b])

        pltpu.sync_copy(slot_hbm, idx_v)
        in_copy(0, 0).start()
        for c in range(total):
            b = c % _NBUF
            f = c + 1
            if f < total:
                fb = f % _NBUF
                if f >= _NBUF:
                    out_copy(f - _NBUF, fb).wait()
                in_copy(f, fb).start()
            in_copy(c, b).wait()
            out_copy(c, b).start()
        for c in range(total - _NBUF, total):
            out_copy(c, c % _NBUF).wait()

        for ch in range(n_tok // _LANES):
            vec = idx_v[pl.ds(ch * _LANES, _LANES)]
            for lane in range(_LANES):
                i = ch * _LANES + lane
                s = vec[lane]

                @pl.when((s >= base) & (s < base + span))
                def _():
                    pltpu.sync_copy(tok_k_hbm.at[i], kc_out.at[s])
                    pltpu.sync_copy(tok_v_hbm.at[i], vc_out.at[s])

    return body(tok_k, tok_v, slot_mapping, k_cache, v_cache)


def kernel(pos_ids, k_val, v_val, slot_mapping, batch_idx, k_cache, v_cache):
    B, H, S, D = k_val.shape
    tok_k = jnp.transpose(k_val, (0, 2, 1, 3)).reshape(B * S, H, D)
    tok_v = jnp.transpose(v_val, (0, 2, 1, 3)).reshape(B * S, H, D)
    return _paged_update(tok_k, tok_v, slot_mapping, k_cache, v_cache)
